# baseline (speedup 1.0000x reference)
import functools

import jax
import jax.numpy as jnp
from jax import lax
from jax.experimental import pallas as pl
from jax.experimental.pallas import tpu as pltpu
from jax.experimental.pallas import tpu_sc as plsc

NC, NS = 2, 16
NW = NC * NS
H = 512
HCH = H // 16
MAX_NB = 32


def _mesh():
    return plsc.VectorSubcoreMesh(
        core_axis_name="c", subcore_axis_name="s",
        num_cores=NC, num_subcores=NS)


def _wid():
    return lax.axis_index("s") * NC + lax.axis_index("c")


def _gather_sum(message, a2b_flat, n_atoms_pad):
    apw = n_atoms_pad // NW

    @functools.partial(
        pl.kernel,
        out_type=jax.ShapeDtypeStruct((n_atoms_pad, H), jnp.float32),
        mesh=_mesh(),
        scratch_types=[
            pltpu.VMEM((apw * MAX_NB,), jnp.int32),
            pltpu.VMEM((MAX_NB, H), jnp.float32),
            pltpu.VMEM((H,), jnp.float32),
            pltpu.SemaphoreType.DMA,
        ],
    )
    def k(msg_hbm, idx_hbm, out_hbm, idx_v, rows_v, acc_v, sem):
        w = _wid()
        abase = w * apw
        pltpu.sync_copy(idx_hbm.at[pl.ds(abase * MAX_NB, apw * MAX_NB)], idx_v)

        def atom(g, carry):
            pltpu.async_copy(
                msg_hbm.at[idx_v.at[pl.ds(g * MAX_NB, MAX_NB)]], rows_v, sem
            ).wait()

            def rstep(r, accs):
                new = []
                for c in range(HCH):
                    a = accs[c]
                    for u in range(4):
                        a = a + rows_v[r * 4 + u, pl.ds(c * 16, 16)]
                    new.append(a)
                return tuple(new)

            accs = lax.fori_loop(
                0, MAX_NB // 4, rstep,
                tuple(jnp.zeros((16,), jnp.float32) for _ in range(HCH)))
            for c in range(HCH):
                acc_v[pl.ds(c * 16, 16)] = accs[c]
            pltpu.sync_copy(acc_v, out_hbm.at[abase + g])
            return carry

        lax.fori_loop(0, apw, atom, 0)

    return k(message, a2b_flat)


def _assemble(a_message_pad, message, b2a, b2revb, n_bonds):
    bpw = n_bonds // NW
    BB = 16
    steps = bpw // BB

    @functools.partial(
        pl.kernel,
        out_type=jax.ShapeDtypeStruct((n_bonds, H), jnp.float32),
        mesh=_mesh(),
        scratch_types=[
            pltpu.VMEM((bpw,), jnp.int32),
            pltpu.VMEM((bpw,), jnp.int32),
            pltpu.VMEM((BB, H), jnp.float32),
            pltpu.VMEM((BB, H), jnp.float32),
            pltpu.SemaphoreType.DMA,
            pltpu.SemaphoreType.DMA,
        ],
    )
    def k(am_hbm, msg_hbm, b2a_hbm, b2r_hbm, out_hbm,
          ia_v, ir_v, arows_v, mrows_v, sem1, sem2):
        w = _wid()
        base = w * bpw
        pltpu.sync_copy(b2a_hbm.at[pl.ds(base, bpw)], ia_v)
        pltpu.sync_copy(b2r_hbm.at[pl.ds(base, bpw)], ir_v)

        def step(s, carry):
            d1 = pltpu.async_copy(
                am_hbm.at[ia_v.at[pl.ds(s * BB, BB)]], arows_v, sem1)
            d2 = pltpu.async_copy(
                msg_hbm.at[ir_v.at[pl.ds(s * BB, BB)]], mrows_v, sem2)
            d1.wait()
            d2.wait()

            def cstep(c, cc):
                for i in range(BB):
                    arows_v[i, pl.ds(c * 16, 16)] = (
                        arows_v[i, pl.ds(c * 16, 16)]
                        - mrows_v[i, pl.ds(c * 16, 16)])
                return cc

            lax.fori_loop(0, HCH, cstep, 0)
            pltpu.sync_copy(arows_v, out_hbm.at[pl.ds(base + s * BB, BB)])
            return carry

        lax.fori_loop(0, steps, step, 0)

    return k(a_message_pad, message, b2a, b2revb)


def _init_body(fb_ref, wi_ref, inp_ref, msg_ref):
    x = lax.dot_general(fb_ref[...], wi_ref[...],
                        (((1,), (1,)), ((), ())),
                        preferred_element_type=jnp.float32)
    inp_ref[...] = x
    msg_ref[...] = jnp.maximum(x, 0.0)


def _depth_body(pre_ref, wh_ref, inp_ref, out_ref):
    x = lax.dot_general(pre_ref[...], wh_ref[...],
                        (((1,), (1,)), ((), ())),
                        preferred_element_type=jnp.float32)
    out_ref[...] = jnp.maximum(inp_ref[...] + x, 0.0)


def _make_out_body(mols_per_blk, rows_per_blk, mol_size):
    def body(fa_ref, am_ref, woa_ref, woh_ref, b_ref, out_ref):
        h = lax.dot_general(fa_ref[...], woa_ref[...],
                            (((1,), (1,)), ((), ())),
                            preferred_element_type=jnp.float32)
        h = h + lax.dot_general(am_ref[...], woh_ref[...],
                                (((1,), (1,)), ((), ())),
                                preferred_element_type=jnp.float32)
        h = jnp.maximum(h + b_ref[...], 0.0)
        r = lax.broadcasted_iota(jnp.int32, (mols_per_blk, rows_per_blk), 1)
        m = lax.broadcasted_iota(jnp.int32, (mols_per_blk, rows_per_blk), 0)
        pool = jnp.where((r >= m * mol_size) & (r < (m + 1) * mol_size),
                         1.0 / mol_size, 0.0).astype(jnp.float32)
        out_ref[...] = lax.dot_general(pool, h, (((1,), (0,)), ((), ())),
                                       preferred_element_type=jnp.float32)
    return body


def kernel(f_atoms, f_bonds, a2b, b2a, b2revb, W_i, W_h, W_o_w, W_o_b):
    n_atoms, atom_fdim = f_atoms.shape
    n_bonds, bond_fdim = f_bonds.shape
    depth = 4
    mol_size = 25
    n_mols = n_atoms // mol_size

    a2b = a2b.astype(jnp.int32)
    b2a = b2a.astype(jnp.int32)
    b2revb = b2revb.astype(jnp.int32)

    apad = ((n_atoms + 8 * NW - 1) // (8 * NW)) * (8 * NW)
    a2b_flat = jnp.zeros((apad, MAX_NB), jnp.int32).at[:n_atoms].set(a2b)
    a2b_flat = a2b_flat.reshape(-1)

    bm = 1280
    inp, message = pl.pallas_call(
        _init_body,
        grid=(n_bonds // bm,),
        in_specs=[pl.BlockSpec((bm, bond_fdim), lambda i: (i, 0)),
                  pl.BlockSpec((H, bond_fdim), lambda i: (0, 0))],
        out_specs=[pl.BlockSpec((bm, H), lambda i: (i, 0)),
                   pl.BlockSpec((bm, H), lambda i: (i, 0))],
        out_shape=[jax.ShapeDtypeStruct((n_bonds, H), jnp.float32),
                   jax.ShapeDtypeStruct((n_bonds, H), jnp.float32)],
    )(f_bonds, W_i)

    for _ in range(depth - 1):
        am_pad = _gather_sum(message, a2b_flat, apad)
        pre = _assemble(am_pad, message, b2a, b2revb, n_bonds)
        message = pl.pallas_call(
            _depth_body,
            grid=(n_bonds // bm,),
            in_specs=[pl.BlockSpec((bm, H), lambda i: (i, 0)),
                      pl.BlockSpec((H, H), lambda i: (0, 0)),
                      pl.BlockSpec((bm, H), lambda i: (i, 0))],
            out_specs=pl.BlockSpec((bm, H), lambda i: (i, 0)),
            out_shape=jax.ShapeDtypeStruct((n_bonds, H), jnp.float32),
        )(pre, W_h, inp)

    am = _gather_sum(message, a2b_flat, apad)[:n_atoms]
    woa = W_o_w[:, :atom_fdim]
    woh = W_o_w[:, atom_fdim:]
    bias = W_o_b.reshape(1, H)

    rows_blk = 2000
    mols_blk = rows_blk // mol_size
    mol_vecs = pl.pallas_call(
        _make_out_body(mols_blk, rows_blk, mol_size),
        grid=(n_atoms // rows_blk,),
        in_specs=[pl.BlockSpec((rows_blk, atom_fdim), lambda i: (i, 0)),
                  pl.BlockSpec((rows_blk, H), lambda i: (i, 0)),
                  pl.BlockSpec((H, atom_fdim), lambda i: (0, 0)),
                  pl.BlockSpec((H, H), lambda i: (0, 0)),
                  pl.BlockSpec((1, H), lambda i: (0, 0))],
        out_specs=pl.BlockSpec((mols_blk, H), lambda i: (i, 0)),
        out_shape=jax.ShapeDtypeStruct((n_mols, H), jnp.float32),
    )(f_atoms, am, woa, woh, bias)
    return mol_vecs


# speedup vs baseline: 1.4846x; 1.4846x over previous
import functools

import jax
import jax.numpy as jnp
from jax import lax
from jax.experimental import pallas as pl
from jax.experimental.pallas import tpu as pltpu
from jax.experimental.pallas import tpu_sc as plsc

NC, NS = 2, 16
NW = NC * NS
H = 512
HCH = H // 16
MAX_NB = 32


def _mesh():
    return plsc.VectorSubcoreMesh(
        core_axis_name="c", subcore_axis_name="s",
        num_cores=NC, num_subcores=NS)


def _wid():
    return lax.axis_index("s") * NC + lax.axis_index("c")


def _gather_sum(message, a2b_flat, n_atoms_pad):
    apw = n_atoms_pad // NW

    @functools.partial(
        pl.kernel,
        out_type=jax.ShapeDtypeStruct((n_atoms_pad, H), jnp.float32),
        mesh=_mesh(),
        scratch_types=[
            pltpu.VMEM((apw * MAX_NB,), jnp.int32),
            pltpu.VMEM((2, MAX_NB, H), jnp.float32),
            pltpu.VMEM((2, H), jnp.float32),
            pltpu.SemaphoreType.DMA,
            pltpu.SemaphoreType.DMA,
            pltpu.SemaphoreType.DMA,
            pltpu.SemaphoreType.DMA,
        ],
    )
    def k(msg_hbm, idx_hbm, out_hbm, idx_v, rows_v, acc_v,
          gsem0, gsem1, wsem0, wsem1):
        w = _wid()
        abase = w * apw
        gsems = (gsem0, gsem1)
        wsems = (wsem0, wsem1)
        pltpu.sync_copy(idx_hbm.at[pl.ds(abase * MAX_NB, apw * MAX_NB)], idx_v)

        def issue_gather(a, slot):
            pltpu.async_copy(
                msg_hbm.at[idx_v.at[pl.ds(a * MAX_NB, MAX_NB)]],
                rows_v.at[slot], gsems[slot])

        issue_gather(0, 0)

        def pair(t, carry):
            for b in range(2):
                a = 2 * t + b
                slot, nslot = b, 1 - b

                @pl.when(a + 1 < apw)
                def _():
                    issue_gather(a + 1, nslot)

                pltpu.make_async_copy(
                    msg_hbm.at[idx_v.at[pl.ds(a * MAX_NB, MAX_NB)]],
                    rows_v.at[slot], gsems[slot]).wait()

                def rstep(r, accs):
                    new = []
                    for c in range(HCH):
                        acc = accs[c]
                        for u in range(4):
                            acc = acc + rows_v[slot, r * 4 + u,
                                               pl.ds(c * 16, 16)]
                        new.append(acc)
                    return tuple(new)

                accs = lax.fori_loop(
                    0, MAX_NB // 4, rstep,
                    tuple(jnp.zeros((16,), jnp.float32) for _ in range(HCH)))

                @pl.when(a >= 2)
                def _():
                    pltpu.make_async_copy(
                        acc_v.at[slot], out_hbm.at[abase + a - 2],
                        wsems[slot]).wait()

                for c in range(HCH):
                    acc_v[slot, pl.ds(c * 16, 16)] = accs[c]
                pltpu.async_copy(acc_v.at[slot], out_hbm.at[abase + a],
                                 wsems[slot])
            return carry

        lax.fori_loop(0, apw // 2, pair, 0)
        for slot in range(2):
            pltpu.make_async_copy(
                acc_v.at[slot], out_hbm.at[abase + apw - 2 + slot],
                wsems[slot]).wait()

    return k(message, a2b_flat)


def _assemble(a_message_pad, message, b2a, b2revb, n_bonds):
    bpw = n_bonds // NW
    BB = 40
    steps = bpw // BB

    @functools.partial(
        pl.kernel,
        out_type=jax.ShapeDtypeStruct((n_bonds, H), jnp.float32),
        mesh=_mesh(),
        scratch_types=[
            pltpu.VMEM((bpw,), jnp.int32),
            pltpu.VMEM((bpw,), jnp.int32),
            pltpu.VMEM((2, BB, H), jnp.float32),
            pltpu.VMEM((2, BB, H), jnp.float32),
            pltpu.SemaphoreType.DMA,
            pltpu.SemaphoreType.DMA,
            pltpu.SemaphoreType.DMA,
            pltpu.SemaphoreType.DMA,
            pltpu.SemaphoreType.DMA,
            pltpu.SemaphoreType.DMA,
        ],
    )
    def k(am_hbm, msg_hbm, b2a_hbm, b2r_hbm, out_hbm,
          ia_v, ir_v, arows_v, mrows_v,
          asem0, asem1, msem0, msem1, wsem0, wsem1):
        w = _wid()
        base = w * bpw
        asems = (asem0, asem1)
        msems = (msem0, msem1)
        wsems = (wsem0, wsem1)
        pltpu.sync_copy(b2a_hbm.at[pl.ds(base, bpw)], ia_v)
        pltpu.sync_copy(b2r_hbm.at[pl.ds(base, bpw)], ir_v)

        def issue_gathers(s, slot):
            pltpu.async_copy(
                am_hbm.at[ia_v.at[pl.ds(s * BB, BB)]],
                arows_v.at[slot], asems[slot])
            pltpu.async_copy(
                msg_hbm.at[ir_v.at[pl.ds(s * BB, BB)]],
                mrows_v.at[slot], msems[slot])

        issue_gathers(0, 0)

        def pair(t, carry):
            for b in range(2):
                s = 2 * t + b
                slot, nslot = b, 1 - b

                @pl.when(s + 1 < steps)
                def _():
                    @pl.when(s >= 1)
                    def _():
                        pltpu.make_async_copy(
                            arows_v.at[nslot],
                            out_hbm.at[pl.ds(base + (s - 1) * BB, BB)],
                            wsems[nslot]).wait()
                    issue_gathers(s + 1, nslot)

                pltpu.make_async_copy(
                    am_hbm.at[ia_v.at[pl.ds(s * BB, BB)]],
                    arows_v.at[slot], asems[slot]).wait()
                pltpu.make_async_copy(
                    msg_hbm.at[ir_v.at[pl.ds(s * BB, BB)]],
                    mrows_v.at[slot], msems[slot]).wait()

                def rstep(i, cc):
                    for c in range(HCH):
                        arows_v[slot, i, pl.ds(c * 16, 16)] = (
                            arows_v[slot, i, pl.ds(c * 16, 16)]
                            - mrows_v[slot, i, pl.ds(c * 16, 16)])
                    return cc

                lax.fori_loop(0, BB, rstep, 0)
                pltpu.async_copy(
                    arows_v.at[slot],
                    out_hbm.at[pl.ds(base + s * BB, BB)], wsems[slot])
            return carry

        lax.fori_loop(0, steps // 2, pair, 0)
        for slot in range(2):
            pltpu.make_async_copy(
                arows_v.at[slot],
                out_hbm.at[pl.ds(base + (steps - 2 + slot) * BB, BB)],
                wsems[slot]).wait()

    return k(a_message_pad, message, b2a, b2revb)


def _init_body(fb_ref, wi_ref, inp_ref, msg_ref):
    x = lax.dot_general(fb_ref[...], wi_ref[...],
                        (((1,), (1,)), ((), ())),
                        preferred_element_type=jnp.float32)
    inp_ref[...] = x
    msg_ref[...] = jnp.maximum(x, 0.0)


def _depth_body(pre_ref, wh_ref, inp_ref, out_ref):
    x = lax.dot_general(pre_ref[...], wh_ref[...],
                        (((1,), (1,)), ((), ())),
                        preferred_element_type=jnp.float32)
    out_ref[...] = jnp.maximum(inp_ref[...] + x, 0.0)


def _make_out_body(mols_per_blk, rows_per_blk, mol_size):
    def body(fa_ref, am_ref, woa_ref, woh_ref, b_ref, out_ref):
        h = lax.dot_general(fa_ref[...], woa_ref[...],
                            (((1,), (1,)), ((), ())),
                            preferred_element_type=jnp.float32)
        h = h + lax.dot_general(am_ref[...], woh_ref[...],
                                (((1,), (1,)), ((), ())),
                                preferred_element_type=jnp.float32)
        h = jnp.maximum(h + b_ref[...], 0.0)
        r = lax.broadcasted_iota(jnp.int32, (mols_per_blk, rows_per_blk), 1)
        m = lax.broadcasted_iota(jnp.int32, (mols_per_blk, rows_per_blk), 0)
        pool = jnp.where((r >= m * mol_size) & (r < (m + 1) * mol_size),
                         1.0 / mol_size, 0.0).astype(jnp.float32)
        out_ref[...] = lax.dot_general(pool, h, (((1,), (0,)), ((), ())),
                                       preferred_element_type=jnp.float32)
    return body


def kernel(f_atoms, f_bonds, a2b, b2a, b2revb, W_i, W_h, W_o_w, W_o_b):
    n_atoms, atom_fdim = f_atoms.shape
    n_bonds, bond_fdim = f_bonds.shape
    depth = 4
    mol_size = 25
    n_mols = n_atoms // mol_size

    a2b = a2b.astype(jnp.int32)
    b2a = b2a.astype(jnp.int32)
    b2revb = b2revb.astype(jnp.int32)

    apad = ((n_atoms + 8 * NW - 1) // (8 * NW)) * (8 * NW)
    a2b_flat = jnp.zeros((apad, MAX_NB), jnp.int32).at[:n_atoms].set(a2b)
    a2b_flat = a2b_flat.reshape(-1)

    bm = 1280
    inp, message = pl.pallas_call(
        _init_body,
        grid=(n_bonds // bm,),
        in_specs=[pl.BlockSpec((bm, bond_fdim), lambda i: (i, 0)),
                  pl.BlockSpec((H, bond_fdim), lambda i: (0, 0))],
        out_specs=[pl.BlockSpec((bm, H), lambda i: (i, 0)),
                   pl.BlockSpec((bm, H), lambda i: (i, 0))],
        out_shape=[jax.ShapeDtypeStruct((n_bonds, H), jnp.float32),
                   jax.ShapeDtypeStruct((n_bonds, H), jnp.float32)],
    )(f_bonds, W_i)

    for _ in range(depth - 1):
        am_pad = _gather_sum(message, a2b_flat, apad)
        pre = _assemble(am_pad, message, b2a, b2revb, n_bonds)
        message = pl.pallas_call(
            _depth_body,
            grid=(n_bonds // bm,),
            in_specs=[pl.BlockSpec((bm, H), lambda i: (i, 0)),
                      pl.BlockSpec((H, H), lambda i: (0, 0)),
                      pl.BlockSpec((bm, H), lambda i: (i, 0))],
            out_specs=pl.BlockSpec((bm, H), lambda i: (i, 0)),
            out_shape=jax.ShapeDtypeStruct((n_bonds, H), jnp.float32),
        )(pre, W_h, inp)

    am = _gather_sum(message, a2b_flat, apad)[:n_atoms]
    woa = W_o_w[:, :atom_fdim]
    woh = W_o_w[:, atom_fdim:]
    bias = W_o_b.reshape(1, H)

    rows_blk = 2000
    mols_blk = rows_blk // mol_size
    mol_vecs = pl.pallas_call(
        _make_out_body(mols_blk, rows_blk, mol_size),
        grid=(n_atoms // rows_blk,),
        in_specs=[pl.BlockSpec((rows_blk, atom_fdim), lambda i: (i, 0)),
                  pl.BlockSpec((rows_blk, H), lambda i: (i, 0)),
                  pl.BlockSpec((H, atom_fdim), lambda i: (0, 0)),
                  pl.BlockSpec((H, H), lambda i: (0, 0)),
                  pl.BlockSpec((1, H), lambda i: (0, 0))],
        out_specs=pl.BlockSpec((mols_blk, H), lambda i: (i, 0)),
        out_shape=jax.ShapeDtypeStruct((n_mols, H), jnp.float32),
    )(f_atoms, am, woa, woh, bias)
    return mol_vecs
